# Initial kernel scaffold; baseline (speedup 1.0000x reference)
import jax
import jax.numpy as jnp
from jax.experimental import pallas as pl


def kernel(doc, emb_table, W1, b1, Uw, W2, b2, Us, Wd, bd):
    raise NotImplementedError("write your pallas kernel here")


# speedup vs baseline: 7.1557x; 7.1557x over previous
import functools

import jax
import jax.numpy as jnp
from jax import lax
from jax.experimental import pallas as pl
from jax.experimental.pallas import tpu as pltpu
from jax.experimental.pallas import tpu_sc as plsc

B, T, V, D, H, L = 16, 2048, 1000000, 128, 128, 50
S, NS = 64, 32
BT = B * T

_NC, _NSUB = 2, 16
NW = _NC * _NSUB
ROWS_W = BT // NW
CH = 128
NCHUNK = ROWS_W // CH


def _sc_gather(table, idx):
    mesh = plsc.VectorSubcoreMesh(core_axis_name="c", subcore_axis_name="s")

    @functools.partial(
        pl.kernel, mesh=mesh,
        out_type=jax.ShapeDtypeStruct((BT, D), jnp.float32),
        scratch_types=[
            pltpu.VMEM((CH,), jnp.int32),
            pltpu.VMEM((CH, D), jnp.float32),
            pltpu.SemaphoreType.DMA,
        ],
    )
    def gk(table_hbm, idx_hbm, out_hbm, idx_v, rows_v, sem):
        wid = lax.axis_index("s") * _NC + lax.axis_index("c")
        base0 = wid * ROWS_W
        for c in range(NCHUNK):
            base = base0 + c * CH
            pltpu.sync_copy(idx_hbm.at[pl.ds(base, CH)], idx_v)
            pltpu.async_copy(table_hbm.at[idx_v], rows_v, sem).wait()
            pltpu.sync_copy(rows_v, out_hbm.at[pl.ds(base, CH)])

    return gk(table, idx)


def _dense_body(x_ref, W1_ref, b1_ref, UwT_ref, W2_ref, b2_ref, Us_ref,
                Wd_ref, bd_ref, out_ref, sent_ref):
    xb = x_ref[0]
    h = jnp.dot(xb, W1_ref[...], preferred_element_type=jnp.float32)
    h = h + b1_ref[...]
    u = jnp.tanh(h)
    ws = jnp.dot(u, UwT_ref[...], preferred_element_type=jnp.float32)
    cols = []
    for n in range(NS):
        wsn = ws[n * S:(n + 1) * S, :]
        hn = h[n * S:(n + 1) * S, :]
        m = jnp.max(wsn, axis=0, keepdims=True)
        e = jnp.exp(wsn - m)
        a = e / jnp.sum(e, axis=0, keepdims=True)
        sent_n = lax.dot_general(a, hn, (((0,), (0,)), ((), ())),
                                 preferred_element_type=jnp.float32)
        v = jnp.tanh(jnp.dot(sent_n, W2_ref[...],
                             preferred_element_type=jnp.float32) + b2_ref[...])
        ss = jnp.sum(v * Us_ref[...], axis=1, keepdims=True)
        sent_ref[n] = sent_n
        cols.append(ss)
    sst = jnp.concatenate(cols, axis=1)
    m2 = jnp.max(sst, axis=1, keepdims=True)
    e2 = jnp.exp(sst - m2)
    sal = e2 / jnp.sum(e2, axis=1, keepdims=True)
    acc = jnp.zeros((L, H), jnp.float32)
    for n in range(NS):
        acc = acc + sent_ref[n] * sal[:, n:n + 1]
    lg = jnp.sum(acc * Wd_ref[...], axis=1)
    b = pl.program_id(0)
    out_ref[pl.ds(b, 1), :] = lg[None, :] + bd_ref[...]


def _tc_dense(x, W1, b1, UwT, W2, b2, Us, Wd, bd):
    return pl.pallas_call(
        _dense_body,
        grid=(B,),
        in_specs=[
            pl.BlockSpec((1, T, D), lambda b: (b, 0, 0)),
            pl.BlockSpec((D, H), lambda b: (0, 0)),
            pl.BlockSpec((1, H), lambda b: (0, 0)),
            pl.BlockSpec((H, L), lambda b: (0, 0)),
            pl.BlockSpec((H, H), lambda b: (0, 0)),
            pl.BlockSpec((1, H), lambda b: (0, 0)),
            pl.BlockSpec((L, H), lambda b: (0, 0)),
            pl.BlockSpec((L, H), lambda b: (0, 0)),
            pl.BlockSpec((1, L), lambda b: (0, 0)),
        ],
        out_specs=pl.BlockSpec((B, L), lambda b: (0, 0)),
        out_shape=jax.ShapeDtypeStruct((B, L), jnp.float32),
        scratch_shapes=[pltpu.VMEM((NS, L, H), jnp.float32)],
    )(x, W1, b1, UwT, W2, b2, Us, Wd, bd)


def kernel(doc, emb_table, W1, b1, Uw, W2, b2, Us, Wd, bd):
    idx = doc.reshape(-1).astype(jnp.int32)
    emb = _sc_gather(emb_table, idx)
    x = emb.reshape(B, T, D)
    return _tc_dense(x, W1, b1.reshape(1, H), Uw.T, W2,
                     b2.reshape(1, H), Us, Wd, bd.reshape(1, L))


# speedup vs baseline: 10.0253x; 1.4010x over previous
import functools

import jax
import jax.numpy as jnp
from jax import lax
from jax.experimental import pallas as pl
from jax.experimental.pallas import tpu as pltpu
from jax.experimental.pallas import tpu_sc as plsc

B, T, V, D, H, L = 16, 2048, 1000000, 128, 128, 50
S, NS = 64, 32
LP = 64
BT = B * T

_NC, _NSUB = 2, 16
NW = _NC * _NSUB
ROWS_W = BT // NW
CH = 128
K = 2
G = ROWS_W // (CH * K)


def _sc_gather(table, idx):
    mesh = plsc.VectorSubcoreMesh(core_axis_name="c", subcore_axis_name="s")

    @functools.partial(
        pl.kernel, mesh=mesh,
        out_type=jax.ShapeDtypeStruct((BT, D), jnp.float32),
        scratch_types=[
            pltpu.VMEM((2 * K, CH), jnp.int32),
            pltpu.VMEM((2 * K, CH, D), jnp.float32),
            pltpu.SemaphoreType.DMA,
            pltpu.SemaphoreType.DMA,
            pltpu.SemaphoreType.DMA,
            pltpu.SemaphoreType.DMA,
        ],
    )
    def gk(table_hbm, idx_hbm, out_hbm, idx_v, rows_v, gs0, gs1, os0, os1):
        wid = lax.axis_index("s") * _NC + lax.axis_index("c")
        base0 = wid * ROWS_W
        gsems = (gs0, gs1)
        osems = (os0, os1)

        def fire_gather(g):
            bs = g % 2
            descs = []
            for j in range(K):
                base = base0 + (g * K + j) * CH
                slot = bs * K + j
                pltpu.sync_copy(idx_hbm.at[pl.ds(base, CH)], idx_v.at[slot])
                descs.append(pltpu.async_copy(
                    table_hbm.at[idx_v.at[slot]], rows_v.at[slot], gsems[bs]))
            return descs

        gd = {0: fire_gather(0)}
        od = {}
        for g in range(G):
            bs = g % 2
            if g >= 1:
                for d in od.pop(g - 1):
                    d.wait()
            if g + 1 < G:
                gd[g + 1] = fire_gather(g + 1)
            for d in gd.pop(g):
                d.wait()
            outs = []
            for j in range(K):
                base = base0 + (g * K + j) * CH
                slot = bs * K + j
                outs.append(pltpu.async_copy(
                    rows_v.at[slot], out_hbm.at[pl.ds(base, CH)], osems[bs]))
            od[g] = outs
        for d in od.pop(G - 1):
            d.wait()

    return gk(table, idx)


def _dense_body(x_ref, W1_ref, b1_ref, UwT_ref, W2_ref, b2_ref, Us_ref,
                Wd_ref, bd_ref, out_ref):
    xb = x_ref[0]
    h = jnp.dot(xb, W1_ref[...], preferred_element_type=jnp.float32)
    h = h + b1_ref[...]
    u = jnp.tanh(h)
    ws = jnp.dot(u, UwT_ref[...], preferred_element_type=jnp.float32)
    ws3 = ws.reshape(NS, S, LP)
    m3 = jnp.max(ws3, axis=1, keepdims=True)
    e3 = jnp.exp(ws3 - m3)
    a3 = e3 / jnp.sum(e3, axis=1, keepdims=True)
    a = a3.reshape(T, LP)
    sent_parts = []
    for n in range(NS):
        an = a[n * S:(n + 1) * S, :]
        hn = h[n * S:(n + 1) * S, :]
        sent_parts.append(lax.dot_general(
            an, hn, (((0,), (0,)), ((), ())),
            preferred_element_type=jnp.float32))
    sent_all = jnp.concatenate(sent_parts, axis=0)
    v = jnp.tanh(jnp.dot(sent_all, W2_ref[...],
                         preferred_element_type=jnp.float32) + b2_ref[...])
    p3 = v.reshape(NS, LP, H) * Us_ref[...][None, :, :]
    ss2 = jnp.sum(p3, axis=2)
    m2 = jnp.max(ss2, axis=0, keepdims=True)
    e2 = jnp.exp(ss2 - m2)
    sal = e2 / jnp.sum(e2, axis=0, keepdims=True)
    sent3 = sent_all.reshape(NS, LP, H)
    dr = jnp.sum(sent3 * sal[:, :, None], axis=0)
    lg = jnp.sum(dr * Wd_ref[...], axis=1)
    b = pl.program_id(0)
    out_ref[pl.ds(b, 1), :] = lg[:L][None, :] + bd_ref[...]


def _tc_dense(x, W1, b1, UwT, W2, b2, Us, Wd, bd):
    return pl.pallas_call(
        _dense_body,
        grid=(B,),
        in_specs=[
            pl.BlockSpec((1, T, D), lambda b: (b, 0, 0)),
            pl.BlockSpec((D, H), lambda b: (0, 0)),
            pl.BlockSpec((1, H), lambda b: (0, 0)),
            pl.BlockSpec((H, LP), lambda b: (0, 0)),
            pl.BlockSpec((H, H), lambda b: (0, 0)),
            pl.BlockSpec((1, H), lambda b: (0, 0)),
            pl.BlockSpec((LP, H), lambda b: (0, 0)),
            pl.BlockSpec((LP, H), lambda b: (0, 0)),
            pl.BlockSpec((1, L), lambda b: (0, 0)),
        ],
        out_specs=pl.BlockSpec((B, L), lambda b: (0, 0)),
        out_shape=jax.ShapeDtypeStruct((B, L), jnp.float32),
    )(x, W1, b1, UwT, W2, b2, Us, Wd, bd)


def kernel(doc, emb_table, W1, b1, Uw, W2, b2, Us, Wd, bd):
    idx = doc.reshape(-1).astype(jnp.int32)
    emb = _sc_gather(emb_table, idx)
    x = emb.reshape(B, T, D)
    pad = ((0, LP - L), (0, 0))
    return _tc_dense(x, W1, b1.reshape(1, H), jnp.pad(Uw, pad).T, W2,
                     b2.reshape(1, H), jnp.pad(Us, pad), jnp.pad(Wd, pad),
                     bd.reshape(1, L))


# speedup vs baseline: 10.4465x; 1.0420x over previous
import functools

import jax
import jax.numpy as jnp
from jax import lax
from jax.experimental import pallas as pl
from jax.experimental.pallas import tpu as pltpu
from jax.experimental.pallas import tpu_sc as plsc

B, T, V, D, H, L = 16, 2048, 1000000, 128, 128, 50
S, NS = 64, 32
LP = 64
BT = B * T

_NC, _NSUB = 2, 16
NW = _NC * _NSUB
NCK = 2
CB = B // NCK
BTC = CB * T
ROWS_W = BTC // NW
CH = 128
K = 2
G = ROWS_W // (CH * K)


def _sc_gather(table, idx):
    mesh = plsc.VectorSubcoreMesh(core_axis_name="c", subcore_axis_name="s")

    @functools.partial(
        pl.kernel, mesh=mesh,
        out_type=jax.ShapeDtypeStruct((BTC, D), jnp.float32),
        scratch_types=[
            pltpu.VMEM((2 * K, CH), jnp.int32),
            pltpu.VMEM((2 * K, CH, D), jnp.float32),
            pltpu.SemaphoreType.DMA,
            pltpu.SemaphoreType.DMA,
            pltpu.SemaphoreType.DMA,
            pltpu.SemaphoreType.DMA,
        ],
    )
    def gk(table_hbm, idx_hbm, out_hbm, idx_v, rows_v, gs0, gs1, os0, os1):
        wid = lax.axis_index("s") * _NC + lax.axis_index("c")
        base0 = wid * ROWS_W
        gsems = (gs0, gs1)
        osems = (os0, os1)

        def fire_gather(g):
            bs = g % 2
            descs = []
            for j in range(K):
                base = base0 + (g * K + j) * CH
                slot = bs * K + j
                pltpu.sync_copy(idx_hbm.at[pl.ds(base, CH)], idx_v.at[slot])
                descs.append(pltpu.async_copy(
                    table_hbm.at[idx_v.at[slot]], rows_v.at[slot], gsems[bs]))
            return descs

        gd = {0: fire_gather(0)}
        od = {}
        for g in range(G):
            bs = g % 2
            if g >= 1:
                for d in od.pop(g - 1):
                    d.wait()
            if g + 1 < G:
                gd[g + 1] = fire_gather(g + 1)
            for d in gd.pop(g):
                d.wait()
            outs = []
            for j in range(K):
                base = base0 + (g * K + j) * CH
                slot = bs * K + j
                outs.append(pltpu.async_copy(
                    rows_v.at[slot], out_hbm.at[pl.ds(base, CH)], osems[bs]))
            od[g] = outs
        for d in od.pop(G - 1):
            d.wait()

    return gk(table, idx)


def _dense_body(x_ref, W1_ref, b1_ref, UwT_ref, W2_ref, b2_ref, Us_ref,
                Wd_ref, bd_ref, out_ref):
    xb = x_ref[0]
    h = jnp.dot(xb, W1_ref[...], preferred_element_type=jnp.float32)
    h = h + b1_ref[...]
    u = jnp.tanh(h)
    ws = jnp.dot(u, UwT_ref[...], preferred_element_type=jnp.float32)
    ws3 = ws.reshape(NS, S, LP)
    m3 = jnp.max(ws3, axis=1, keepdims=True)
    e3 = jnp.exp(ws3 - m3)
    a3 = e3 / jnp.sum(e3, axis=1, keepdims=True)
    a = a3.reshape(T, LP)
    sent_parts = []
    for n in range(NS):
        an = a[n * S:(n + 1) * S, :]
        hn = h[n * S:(n + 1) * S, :]
        sent_parts.append(lax.dot_general(
            an, hn, (((0,), (0,)), ((), ())),
            preferred_element_type=jnp.float32))
    sent_all = jnp.concatenate(sent_parts, axis=0)
    v = jnp.tanh(jnp.dot(sent_all, W2_ref[...],
                         preferred_element_type=jnp.float32) + b2_ref[...])
    p3 = v.reshape(NS, LP, H) * Us_ref[...][None, :, :]
    ss2 = jnp.sum(p3, axis=2)
    m2 = jnp.max(ss2, axis=0, keepdims=True)
    e2 = jnp.exp(ss2 - m2)
    sal = e2 / jnp.sum(e2, axis=0, keepdims=True)
    sent3 = sent_all.reshape(NS, LP, H)
    dr = jnp.sum(sent3 * sal[:, :, None], axis=0)
    lg = jnp.sum(dr * Wd_ref[...], axis=1)
    b = pl.program_id(0)
    out_ref[pl.ds(b, 1), :] = lg[:L][None, :] + bd_ref[...]


def _tc_dense(x, W1, b1, UwT, W2, b2, Us, Wd, bd):
    return pl.pallas_call(
        _dense_body,
        grid=(CB,),
        in_specs=[
            pl.BlockSpec((1, T, D), lambda b: (b, 0, 0)),
            pl.BlockSpec((D, H), lambda b: (0, 0)),
            pl.BlockSpec((1, H), lambda b: (0, 0)),
            pl.BlockSpec((H, LP), lambda b: (0, 0)),
            pl.BlockSpec((H, H), lambda b: (0, 0)),
            pl.BlockSpec((1, H), lambda b: (0, 0)),
            pl.BlockSpec((LP, H), lambda b: (0, 0)),
            pl.BlockSpec((LP, H), lambda b: (0, 0)),
            pl.BlockSpec((1, L), lambda b: (0, 0)),
        ],
        out_specs=pl.BlockSpec((CB, L), lambda b: (0, 0)),
        out_shape=jax.ShapeDtypeStruct((CB, L), jnp.float32),
    )(x, W1, b1, UwT, W2, b2, Us, Wd, bd)


def kernel(doc, emb_table, W1, b1, Uw, W2, b2, Us, Wd, bd):
    idx = doc.reshape(-1).astype(jnp.int32)
    pad = ((0, LP - L), (0, 0))
    b1r, b2r, bdr = b1.reshape(1, H), b2.reshape(1, H), bd.reshape(1, L)
    UwTp, Usp, Wdp = jnp.pad(Uw, pad).T, jnp.pad(Us, pad), jnp.pad(Wd, pad)
    outs = []
    for c in range(NCK):
        emb = _sc_gather(table=emb_table, idx=idx[c * BTC:(c + 1) * BTC])
        x = emb.reshape(CB, T, D)
        outs.append(_tc_dense(x, W1, b1r, UwTp, W2, b2r, Usp, Wdp, bdr))
    return jnp.concatenate(outs, axis=0)
